# Initial kernel scaffold; baseline (speedup 1.0000x reference)
import functools

import jax
import jax.numpy as jnp
from jax import lax
from jax.experimental import pallas as pl
from jax.experimental.pallas import tpu as pltpu
from jax.experimental.pallas import tpu_sc as plsc

D = 64
NW = 32
CHUNK = 128
NBUF = 4


def _build(B):
    b_w = B // NW
    nch = b_w // CHUNK
    mesh = plsc.VectorSubcoreMesh(core_axis_name="c", subcore_axis_name="s")

    @functools.partial(
        pl.kernel,
        mesh=mesh,
        out_type=jax.ShapeDtypeStruct((B, D), jnp.float32),
        scratch_types=[
            pltpu.VMEM((nch, CHUNK), jnp.int32),
            [pltpu.VMEM((CHUNK, D), jnp.float32) for _ in range(NBUF)],
            pltpu.SemaphoreType.DMA,
            pltpu.SemaphoreType.DMA,
        ],
    )
    def kern(idx_hbm, table_hbm, out_hbm, idx_v, bufs, gsem, osem):
        cid = lax.axis_index("c")
        sid = lax.axis_index("s")
        wid = sid * 2 + cid
        base = wid * b_w
        pltpu.sync_copy(idx_hbm.at[wid], idx_v)

        def group(g, _):
            j0 = g * NBUF
            gets = [
                pltpu.async_copy(
                    table_hbm.at[idx_v.at[j0 + b]], bufs[b], gsem
                )
                for b in range(NBUF)
            ]
            for c in gets:
                c.wait()
            puts = [
                pltpu.async_copy(
                    bufs[b],
                    out_hbm.at[pl.ds(base + (j0 + b) * CHUNK, CHUNK)],
                    osem,
                )
                for b in range(NBUF)
            ]
            for c in puts:
                c.wait()
            return 0

        lax.fori_loop(0, nch // NBUF, group, 0)

    return kern


def kernel(token_ids, weight):
    s0, s1 = token_ids.shape
    B = s0 * s1
    idx = token_ids.reshape(NW, (B // NW) // CHUNK, CHUNK).astype(jnp.int32)
    out = _build(B)(idx, weight)
    return out.reshape(s0, s1, D)


# speedup vs baseline: 1.8291x; 1.8291x over previous
import functools

import jax
import jax.numpy as jnp
from jax import lax
from jax.experimental import pallas as pl
from jax.experimental.pallas import tpu as pltpu
from jax.experimental.pallas import tpu_sc as plsc

D = 64
NW = 32
CHUNK = 128
NBUF = 4


def _build(B):
    b_w = B // NW
    nch = b_w // CHUNK
    mesh = plsc.VectorSubcoreMesh(core_axis_name="c", subcore_axis_name="s")

    @functools.partial(
        pl.kernel,
        mesh=mesh,
        out_type=jax.ShapeDtypeStruct((B, D), jnp.float32),
        scratch_types=[
            pltpu.VMEM((nch, CHUNK), jnp.int32),
            [pltpu.VMEM((CHUNK, D), jnp.float32) for _ in range(NBUF)],
            pltpu.SemaphoreType.DMA,
            pltpu.SemaphoreType.DMA,
        ],
        compiler_params=pltpu.CompilerParams(use_tc_tiling_on_sc=False),
    )
    def kern(idx_hbm, table_hbm, out_hbm, idx_v, bufs, gsem, osem):
        cid = lax.axis_index("c")
        sid = lax.axis_index("s")
        wid = sid * 2 + cid
        base = wid * b_w
        pltpu.sync_copy(idx_hbm.at[wid], idx_v)

        def group(g, _):
            j0 = g * NBUF
            gets = [
                pltpu.async_copy(
                    table_hbm.at[idx_v.at[j0 + b]], bufs[b], gsem
                )
                for b in range(NBUF)
            ]
            for c in gets:
                c.wait()
            puts = [
                pltpu.async_copy(
                    bufs[b],
                    out_hbm.at[pl.ds(base + (j0 + b) * CHUNK, CHUNK)],
                    osem,
                )
                for b in range(NBUF)
            ]
            for c in puts:
                c.wait()
            return 0

        lax.fori_loop(0, nch // NBUF, group, 0)

    return kern


def kernel(token_ids, weight):
    s0, s1 = token_ids.shape
    B = s0 * s1
    idx = token_ids.reshape(NW, (B // NW) // CHUNK, CHUNK).astype(jnp.int32)
    out = _build(B)(idx, weight)
    return out.reshape(s0, s1, D)


# speedup vs baseline: 1.8698x; 1.0223x over previous
import functools

import jax
import jax.numpy as jnp
from jax import lax
from jax.experimental import pallas as pl
from jax.experimental.pallas import tpu as pltpu
from jax.experimental.pallas import tpu_sc as plsc

D = 64
NW = 32
CHUNK = 128
NBUF = 8


def _build(B):
    b_w = B // NW
    nch = b_w // CHUNK
    mesh = plsc.VectorSubcoreMesh(core_axis_name="c", subcore_axis_name="s")

    @functools.partial(
        pl.kernel,
        mesh=mesh,
        out_type=jax.ShapeDtypeStruct((B, D), jnp.float32),
        scratch_types=[
            pltpu.VMEM((nch, CHUNK), jnp.int32),
            [pltpu.VMEM((CHUNK, D), jnp.float32) for _ in range(NBUF)],
            [pltpu.SemaphoreType.DMA for _ in range(NBUF)],
            [pltpu.SemaphoreType.DMA for _ in range(NBUF)],
        ],
        compiler_params=pltpu.CompilerParams(use_tc_tiling_on_sc=False),
    )
    def kern(idx_hbm, table_hbm, out_hbm, idx_v, bufs, gs, os):
        cid = lax.axis_index("c")
        sid = lax.axis_index("s")
        wid = sid * 2 + cid
        base = wid * b_w
        pltpu.sync_copy(idx_hbm.at[wid], idx_v)

        for b in range(NBUF):
            pltpu.async_copy(table_hbm.at[idx_v.at[b]], bufs[b], gs[b])

        def cycle(k, _):
            for b in range(NBUF):
                j = k * NBUF + b
                pltpu.make_async_copy(
                    table_hbm.at[idx_v.at[0]], bufs[b], gs[b]
                ).wait()
                pltpu.async_copy(
                    bufs[b], out_hbm.at[pl.ds(base + j * CHUNK, CHUNK)], os[b]
                )
            for b in range(NBUF):
                j = k * NBUF + b
                pltpu.make_async_copy(
                    bufs[b], out_hbm.at[pl.ds(base, CHUNK)], os[b]
                ).wait()

                @pl.when(j + NBUF < nch)
                def _refire():
                    pltpu.async_copy(
                        table_hbm.at[idx_v.at[j + NBUF]], bufs[b], gs[b]
                    )

            return 0

        lax.fori_loop(0, nch // NBUF, cycle, 0)

    return kern


def kernel(token_ids, weight):
    s0, s1 = token_ids.shape
    B = s0 * s1
    idx = token_ids.reshape(NW, (B // NW) // CHUNK, CHUNK).astype(jnp.int32)
    out = _build(B)(idx, weight)
    return out.reshape(s0, s1, D)
